# baseline (speedup 1.0000x reference)
import jax
import jax.numpy as jnp
from jax import lax
from jax.experimental import pallas as pl
from jax.experimental.pallas import tpu as pltpu
import jax.experimental.pallas.tpu_sc as plsc

_L = 16


def _bias_sc_kernel(row_hbm, col_hbm, bias_hbm, row_v, col_v, out_v):
    nc = 2
    wid = lax.axis_index("s") * nc + lax.axis_index("c")
    for t in range(2):
        p = wid * 2 + t
        r = p // 8
        c = p - 8 * r
        pltpu.sync_copy(row_hbm.at[r], row_v)
        pltpu.sync_copy(col_hbm.at[c], col_v)
        for j in range(128 // _L):
            sl = pl.ds(j * _L, _L)
            out_v[sl] = row_v[sl] + col_v[sl]
        pltpu.sync_copy(out_v, bias_hbm.at[p])


def _compute_bias(row_emb, col_emb):
    d = row_emb.shape[1]
    return pl.kernel(
        _bias_sc_kernel,
        out_type=jax.ShapeDtypeStruct((64, d), jnp.float32),
        mesh=plsc.VectorSubcoreMesh(core_axis_name="c", subcore_axis_name="s"),
        scratch_types=[
            pltpu.VMEM((d,), jnp.float32),
            pltpu.VMEM((d,), jnp.float32),
            pltpu.VMEM((d,), jnp.float32),
        ],
    )(row_emb, col_emb)


def _add_bias_kernel(x_ref, b_ref, o_ref):
    o_ref[...] = x_ref[...] + b_ref[...][None]


def kernel(x, row_emb, col_emb):
    n, s, d = x.shape
    bias = _compute_bias(row_emb, col_emb).reshape(8, 8, d)
    x4 = x.reshape(n, 8, 8, d)
    blk = 256
    grid = (n // blk,)
    out = pl.pallas_call(
        _add_bias_kernel,
        grid=grid,
        in_specs=[
            pl.BlockSpec((blk, 8, 8, d), lambda i: (i, 0, 0, 0)),
            pl.BlockSpec((8, 8, d), lambda i: (0, 0, 0)),
        ],
        out_specs=pl.BlockSpec((blk, 8, 8, d), lambda i: (i, 0, 0, 0)),
        out_shape=jax.ShapeDtypeStruct((n, 8, 8, d), x.dtype),
        compiler_params=pltpu.CompilerParams(
            dimension_semantics=("parallel",),
        ),
    )(x4, bias)
    return out.reshape(n, s, d)


# speedup vs baseline: 1.2598x; 1.2598x over previous
import jax
import jax.numpy as jnp
from jax.experimental import pallas as pl
from jax.experimental.pallas import tpu as pltpu


def _add_bias_kernel(x_ref, r_ref, c_ref, o_ref):
    r = r_ref[...]
    c = c_ref[...]
    o_ref[...] = x_ref[...] + (r[None, :, None, :] + c[None, None, :, :])


def kernel(x, row_emb, col_emb):
    n, s, d = x.shape
    x4 = x.reshape(n, 8, 8, d)
    blk = 256
    grid = (n // blk,)
    out = pl.pallas_call(
        _add_bias_kernel,
        grid=grid,
        in_specs=[
            pl.BlockSpec((blk, 8, 8, d), lambda i: (i, 0, 0, 0)),
            pl.BlockSpec((8, d), lambda i: (0, 0)),
            pl.BlockSpec((8, d), lambda i: (0, 0)),
        ],
        out_specs=pl.BlockSpec((blk, 8, 8, d), lambda i: (i, 0, 0, 0)),
        out_shape=jax.ShapeDtypeStruct((n, 8, 8, d), x.dtype),
        compiler_params=pltpu.CompilerParams(
            dimension_semantics=("parallel",),
            vmem_limit_bytes=100 * 1024 * 1024,
        ),
    )(x4, row_emb, col_emb)
    return out.reshape(n, s, d)
